# baseline (speedup 1.0000x reference)
import jax
import jax.numpy as jnp
from jax import lax
from jax.experimental import pallas as pl
from jax.experimental.pallas import tpu as pltpu
from jax.experimental.pallas import tpu_sc as plsc

_NU = 20000
_NB = 30000
_D = 128
_LEAKY = 0.2
_BATCH = 4096

_N = _NU + _NB
_NP = 50176
_DC = 32
_NCH = 4
_NT = 16
_RT = _NP // _NT
_RC = 32
_NRC = _RT // _RC
_ER = 296
_ERP = _NT * _ER + 8
_NBAT = _ER
_SEL = 3 * _BATCH
_SELP = 16384
_SR = _SELP // 128
_SRT = _SR // _NT


def _sc_body(emb4, filt3, comb3, vals2d, sel2d,
             sel_out, ybuf, e1buf,
             acc,
             ic0, ic1, ic2, ic3,
             vb0, vb1, vb2, vb3, gb0, gb1, gb2, gb3, zbuf,
             abuf, bbuf, cbuf, fbuf,
             is0, is1, is2, is3, gs0, gs1, gs2, gs3,
             ss0, ss1, ss2, ss3):
    IC = [ic0, ic1, ic2, ic3]
    VB = [vb0, vb1, vb2, vb3]
    GB = [gb0, gb1, gb2, gb3]
    ISEM = [is0, is1, is2, is3]
    GSEM = [gs0, gs1, gs2, gs3]
    SSEM = [ss0, ss1, ss2, ss3]
    sc = lax.axis_index("c")
    t = lax.axis_index("s")
    r0_tile = t * _RT
    ebase = t * _ER

    zero16 = jnp.zeros((16,), jnp.float32)

    def _zrow(i, _):
        zbuf[i, 0:16] = zero16
        zbuf[i, 16:32] = zero16
        return 0

    lax.fori_loop(0, 64, _zrow, 0, unroll=8)

    def _clear_acc(k, _):
        pltpu.sync_copy(zbuf, acc.at[pl.ds(r0_tile + k * 64, 64)])
        return 0

    lax.fori_loop(0, _RT // 64, _clear_acc, 0)
    plsc.subcore_barrier()

    def _spmm(src, gsel, ssel):
        def fire_idx(s, b):
            br = ebase + b
            pltpu.async_copy(comb3.at[br], IC[s], ISEM[s])
            pltpu.async_copy(vals2d.at[br], VB[s], ISEM[s])

        def wait_idx(s):
            pltpu.make_async_copy(comb3.at[ebase], IC[s], ISEM[s]).wait()
            pltpu.make_async_copy(vals2d.at[ebase], VB[s], ISEM[s]).wait()

        def fire_g(s):
            pltpu.async_copy(src.at[IC[s].at[gsel]], GB[s], GSEM[s])

        def wait_g(s):
            pltpu.make_async_copy(src.at[IC[s].at[gsel]], GB[s],
                                  GSEM[s]).wait()

        def scale(s):
            vb = VB[s]
            gb = GB[s]

            def grp(k, _):
                vv = vb[pl.ds(k * 16, 16)]
                for lane in range(16):
                    e = k * 16 + lane
                    v = vv[lane]
                    gb[e, 0:16] = gb[e, 0:16] * v
                    gb[e, 16:32] = gb[e, 16:32] * v
                return 0

            lax.fori_loop(0, 8, grp, 0)

        def fire_s(s):
            pltpu.async_copy(GB[s], acc.at[IC[s].at[ssel]], SSEM[s],
                             add=True)

        def wait_s(s):
            pltpu.make_async_copy(GB[s], acc.at[IC[s].at[ssel]],
                                  SSEM[s]).wait()

        def stage(u, b, do_wait_s):
            s3 = (u + 3) % 4
            s2_ = (u + 2) % 4
            if do_wait_s:
                wait_s(s3)
            fire_idx(s3, b + 3)
            wait_idx(s2_)
            fire_g(s2_)
            wait_g(u)
            scale(u)
            fire_s(u)

        fire_idx(0, 0)
        fire_idx(1, 1)
        fire_idx(2, 2)
        wait_idx(0)
        fire_g(0)
        wait_idx(1)
        fire_g(1)
        stage(0, 0, False)
        stage(1, 1, True)
        stage(2, 2, True)
        stage(3, 3, True)

        def quad(q, _):
            b = 4 * q
            stage(0, b, True)
            stage(1, b + 1, True)
            stage(2, b + 2, True)
            stage(3, b + 3, True)
            return 0

        lax.fori_loop(1, _NBAT // 4, quad, 0)
        wait_s(3)
        wait_g(0)
        wait_g(1)
        wait_idx(2)

    def _drain(dst, mode, l=0, s0=None, s1=None):
        def chunk(k, _):
            r0 = r0_tile + k * 64
            g = t * 49 + k
            pltpu.sync_copy(acc.at[pl.ds(r0, 64)], abuf)
            if mode == 1:
                pltpu.sync_copy(filt3.at[l, g], fbuf)

                def rowg(g2, _):
                    fv16 = fbuf[pl.ds(g2 * 16, 16)]
                    fv16 = jnp.where(fv16 >= 0.0, fv16, fv16 * _LEAKY)
                    for lane in range(16):
                        i = g2 * 16 + lane
                        fv = fv16[lane]
                        abuf[i, 0:16] = abuf[i, 0:16] * fv
                        abuf[i, 16:32] = abuf[i, 16:32] * fv
                    return 0

                lax.fori_loop(0, 4, rowg, 0)
            elif mode == 2:
                pltpu.sync_copy(s0.at[pl.ds(r0, 64)], bbuf)
                pltpu.sync_copy(s1.at[pl.ds(r0, 64)], cbuf)
                third = jnp.float32(1.0 / 3.0)

                def rowm(i, _):
                    abuf[i, 0:16] = (abuf[i, 0:16] + bbuf[i, 0:16]
                                     + cbuf[i, 0:16]) * third
                    abuf[i, 16:32] = (abuf[i, 16:32] + bbuf[i, 16:32]
                                      + cbuf[i, 16:32]) * third
                    return 0

                lax.fori_loop(0, 64, rowm, 0, unroll=2)
            pltpu.async_copy(abuf, dst.at[pl.ds(r0, 64)], ss0)
            pltpu.async_copy(zbuf, acc.at[pl.ds(r0, 64)], gs0)
            return 0

        def chunk_w(k, _):
            pltpu.make_async_copy(abuf, dst.at[pl.ds(r0_tile, 64)],
                                  ss0).wait()
            pltpu.make_async_copy(zbuf, acc.at[pl.ds(r0_tile, 64)],
                                  gs0).wait()
            chunk(k, 0)
            return 0

        chunk(0, 0)
        lax.fori_loop(1, 49, chunk_w, 0)
        pltpu.make_async_copy(abuf, dst.at[pl.ds(r0_tile, 64)], ss0).wait()
        pltpu.make_async_copy(zbuf, acc.at[pl.ds(r0_tile, 64)], gs0).wait()

    def _round(r, _):
        c = 2 * r + sc
        src0 = emb4.at[c]
        y = ybuf.at[c]
        e1 = e1buf.at[c]
        _spmm(src0, 0, 1)
        plsc.subcore_barrier()
        _drain(y, mode=1, l=0)
        plsc.subcore_barrier()
        _spmm(y, 1, 0)
        plsc.subcore_barrier()
        _drain(e1, mode=0)
        plsc.subcore_barrier()
        _spmm(e1, 0, 1)
        plsc.subcore_barrier()
        _drain(y, mode=1, l=1)
        plsc.subcore_barrier()
        _spmm(y, 1, 0)
        plsc.subcore_barrier()
        _drain(y, mode=2, s0=src0, s1=e1)
        plsc.subcore_barrier()
        s0i = t * _SRT
        for gp in range(2):
            for s in range(4):
                pltpu.async_copy(sel2d.at[s0i + gp * 4 + s], IC[s].at[0],
                                 ISEM[s])
            for s in range(4):
                pltpu.make_async_copy(sel2d.at[s0i], IC[s].at[0],
                                      ISEM[s]).wait()
                pltpu.async_copy(y.at[IC[s].at[0]], GB[s], GSEM[s])
            for s in range(4):
                pltpu.make_async_copy(y.at[IC[s].at[0]], GB[s],
                                      GSEM[s]).wait()
                pltpu.sync_copy(GB[s], sel_out.at[c].at[s0i + gp * 4 + s])
        plsc.subcore_barrier()
        return 0

    lax.fori_loop(0, 2, _round, 0)


def _sc_propagate(emb4, filt3, comb3, vals2d, sel2d):
    mesh = plsc.VectorSubcoreMesh(core_axis_name="c", subcore_axis_name="s")
    f32 = jnp.float32
    out_type = (
        jax.ShapeDtypeStruct((_NCH, _SR, 128, _DC), f32),
        jax.ShapeDtypeStruct((_NCH, _NP, _DC), f32),
        jax.ShapeDtypeStruct((_NCH, _NP, _DC), f32),
    )
    scratch = (
        [pltpu.VMEM_SHARED((_NP, _DC), f32)]
        + [pltpu.VMEM((2, 128), jnp.int32)] * 4
        + [pltpu.VMEM((128,), f32)] * 4
        + [pltpu.VMEM((128, _DC), f32)] * 4
        + [pltpu.VMEM((64, _DC), f32)]
        + [pltpu.VMEM((64, _DC), f32)] * 3
        + [pltpu.VMEM((64,), f32)]
        + [pltpu.SemaphoreType.DMA] * 12
    )
    run = pl.kernel(_sc_body, out_type=out_type, mesh=mesh,
                    scratch_types=scratch,
                    compiler_params=pltpu.CompilerParams(
                        use_tc_tiling_on_sc=False))
    sel4, _, _ = run(emb4, filt3, comb3, vals2d, sel2d)
    return sel4


def _loss_body(ue_ref, bp_ref, bn_ref, out_ref):
    i = pl.program_id(0)

    @pl.when(i == 0)
    def _():
        out_ref[0, 0] = 0.0

    x = jnp.sum(ue_ref[...] * (bn_ref[...] - bp_ref[...]), axis=1)
    sp = jnp.maximum(x, 0.0) + jnp.log1p(jnp.exp(-jnp.abs(x)))
    out_ref[0, 0] += jnp.sum(sp)

    @pl.when(i == pl.num_programs(0) - 1)
    def _():
        out_ref[0, 0] = out_ref[0, 0] * (1.0 / _BATCH)


def _l2_body(e_ref, out_ref):
    i = pl.program_id(0)

    @pl.when(i == 0)
    def _():
        out_ref[0, 0] = 0.0

    x = e_ref[...]
    out_ref[0, 0] += jnp.sum(x * x)

    @pl.when(i == pl.num_programs(0) - 1)
    def _():
        out_ref[0, 0] = out_ref[0, 0] * (0.5 / _NU)


def _tc_loss(ue, bp, bn):
    blk = 512
    grid = _BATCH // blk
    out = pl.pallas_call(
        _loss_body,
        grid=(grid,),
        in_specs=[pl.BlockSpec((blk, _D), lambda i: (i, 0))] * 3,
        out_specs=pl.BlockSpec(memory_space=pltpu.SMEM),
        out_shape=jax.ShapeDtypeStruct((1, 1), jnp.float32),
    )(ue, bp, bn)
    return out[0, 0]


def _tc_l2(all_emb):
    blk = 2000
    grid = _N // blk
    out = pl.pallas_call(
        _l2_body,
        grid=(grid,),
        in_specs=[pl.BlockSpec((blk, _D), lambda i: (i, 0))],
        out_specs=pl.BlockSpec(memory_space=pltpu.SMEM),
        out_shape=jax.ShapeDtypeStruct((1, 1), jnp.float32),
    )(all_emb)
    return out[0, 0]


def kernel(u, b, emb_u, emb_b, filt, graph_rows, graph_cols, graph_vals):
    f32 = jnp.float32
    all_emb = jnp.concatenate([emb_u, emb_b], axis=0)
    emb_pad = jnp.pad(all_emb, ((0, _NP - _N), (0, 0)))
    emb4 = emb_pad.reshape(_NP, _NCH, _DC).transpose(1, 0, 2)

    filt3 = jnp.pad(filt, ((0, 0), (0, _NP - _N))).reshape(
        2, _NP // 64, 64)

    pad_e = _ERP * 128 - graph_rows.shape[0]
    rows2d = jnp.pad(graph_rows, (0, pad_e)).reshape(_ERP, 128)
    cols2d = jnp.pad(graph_cols, (0, pad_e)).reshape(_ERP, 128)
    vals2d = jnp.pad(graph_vals, (0, pad_e)).reshape(_ERP, 128)
    comb3 = jnp.stack([rows2d, cols2d], axis=1)

    sel = jnp.concatenate([u[:, 0], b[:, 0] + _NU, b[:, 1] + _NU])
    sel2d = jnp.pad(sel.astype(jnp.int32), (0, _SELP - _SEL)).reshape(
        _SR, 128)

    sel4 = _sc_propagate(emb4, filt3, comb3, vals2d, sel2d)
    sel_rows = sel4.reshape(_NCH, _SELP, _DC)[:, :_SEL]
    sel_rows = sel_rows.transpose(1, 0, 2).reshape(_SEL, _D)
    ue = sel_rows[:_BATCH]
    bp = sel_rows[_BATCH:2 * _BATCH]
    bn = sel_rows[2 * _BATCH:]

    loss = _tc_loss(ue, bp, bn)
    l2 = _tc_l2(all_emb)
    return (loss.astype(f32), l2.astype(f32))


# speedup vs baseline: 1.0607x; 1.0607x over previous
import jax
import jax.numpy as jnp
from jax import lax
from jax.experimental import pallas as pl
from jax.experimental.pallas import tpu as pltpu
from jax.experimental.pallas import tpu_sc as plsc

_NU = 20000
_NB = 30000
_D = 128
_LEAKY = 0.2
_BATCH = 4096

_N = _NU + _NB
_NP = 50176
_DC = 32
_NCH = 4
_NT = 16
_RT = _NP // _NT
_RC = 32
_NRC = _RT // _RC
_ER = 296
_ERP = _NT * _ER + 8
_NBAT = _ER
_SEL = 3 * _BATCH
_SELP = 16384
_SR = _SELP // 128
_SRT = _SR // _NT


def _sc_body(emb4, filt3, rows2d, cols2d, vals2d, sel2d,
             sel_out, ybuf, e1buf,
             acc,
             gi0, gi1, gi2, gi3, si0, si1, si2, si3,
             vb0, vb1, vb2, vb3, gb0, gb1, gb2, gb3, zbuf,
             abuf, bbuf, cbuf, fbuf,
             is0, is1, is2, is3, gs0, gs1, gs2, gs3,
             ss0, ss1, ss2, ss3):
    GI = [gi0, gi1, gi2, gi3]
    SI = [si0, si1, si2, si3]
    VB = [vb0, vb1, vb2, vb3]
    GB = [gb0, gb1, gb2, gb3]
    ISEM = [is0, is1, is2, is3]
    GSEM = [gs0, gs1, gs2, gs3]
    SSEM = [ss0, ss1, ss2, ss3]
    sc = lax.axis_index("c")
    t = lax.axis_index("s")
    r0_tile = t * _RT
    ebase = t * _ER

    zero16 = jnp.zeros((16,), jnp.float32)

    def _zrow(i, _):
        zbuf[i, 0:16] = zero16
        zbuf[i, 16:32] = zero16
        return 0

    lax.fori_loop(0, 64, _zrow, 0, unroll=8)

    def _clear_acc(k, _):
        pltpu.sync_copy(zbuf, acc.at[pl.ds(r0_tile + k * 64, 64)])
        return 0

    lax.fori_loop(0, _RT // 64, _clear_acc, 0)
    plsc.subcore_barrier()

    def _spmm(src, g2d, s2d):
        def fire_idx(s, b):
            br = ebase + b
            pltpu.async_copy(g2d.at[br], GI[s], ISEM[s])
            pltpu.async_copy(s2d.at[br], SI[s], ISEM[s])
            pltpu.async_copy(vals2d.at[br], VB[s], ISEM[s])

        def wait_idx(s):
            pltpu.make_async_copy(g2d.at[ebase], GI[s], ISEM[s]).wait()
            pltpu.make_async_copy(s2d.at[ebase], SI[s], ISEM[s]).wait()
            pltpu.make_async_copy(vals2d.at[ebase], VB[s], ISEM[s]).wait()

        def fire_g(s):
            pltpu.async_copy(src.at[GI[s]], GB[s], GSEM[s])

        def wait_g(s):
            pltpu.make_async_copy(src.at[GI[s]], GB[s], GSEM[s]).wait()

        def scale(s):
            vb = VB[s]
            gb = GB[s]

            def grp(k, _):
                vv = vb[pl.ds(k * 16, 16)]
                for lane in range(16):
                    e = k * 16 + lane
                    v = vv[lane]
                    gb[e, 0:16] = gb[e, 0:16] * v
                    gb[e, 16:32] = gb[e, 16:32] * v
                return 0

            lax.fori_loop(0, 8, grp, 0)

        def fire_s(s):
            pltpu.async_copy(GB[s], acc.at[SI[s]], SSEM[s], add=True)

        def wait_s(s):
            pltpu.make_async_copy(GB[s], acc.at[SI[s]], SSEM[s]).wait()

        def stage(u, b, do_wait_s):
            s3 = (u + 3) % 4
            s2_ = (u + 2) % 4
            if do_wait_s:
                wait_s(s3)
            fire_idx(s3, b + 3)
            wait_idx(s2_)
            fire_g(s2_)
            wait_g(u)
            scale(u)
            fire_s(u)

        fire_idx(0, 0)
        fire_idx(1, 1)
        fire_idx(2, 2)
        wait_idx(0)
        fire_g(0)
        wait_idx(1)
        fire_g(1)
        stage(0, 0, False)
        stage(1, 1, True)
        stage(2, 2, True)
        stage(3, 3, True)

        def quad(q, _):
            b = 4 * q
            stage(0, b, True)
            stage(1, b + 1, True)
            stage(2, b + 2, True)
            stage(3, b + 3, True)
            return 0

        lax.fori_loop(1, _NBAT // 4, quad, 0)
        wait_s(3)
        wait_g(0)
        wait_g(1)
        wait_idx(2)

    def _drain(dst, mode, l=0, s0=None, s1=None):
        def chunk(k, _):
            r0 = r0_tile + k * 64
            g = t * 49 + k
            pltpu.sync_copy(acc.at[pl.ds(r0, 64)], abuf)
            if mode == 1:
                pltpu.sync_copy(filt3.at[l, g], fbuf)

                def rowg(g2, _):
                    fv16 = fbuf[pl.ds(g2 * 16, 16)]
                    fv16 = jnp.where(fv16 >= 0.0, fv16, fv16 * _LEAKY)
                    for lane in range(16):
                        i = g2 * 16 + lane
                        fv = fv16[lane]
                        abuf[i, 0:16] = abuf[i, 0:16] * fv
                        abuf[i, 16:32] = abuf[i, 16:32] * fv
                    return 0

                lax.fori_loop(0, 4, rowg, 0)
            elif mode == 2:
                pltpu.sync_copy(s0.at[pl.ds(r0, 64)], bbuf)
                pltpu.sync_copy(s1.at[pl.ds(r0, 64)], cbuf)
                third = jnp.float32(1.0 / 3.0)

                def rowm(i, _):
                    abuf[i, 0:16] = (abuf[i, 0:16] + bbuf[i, 0:16]
                                     + cbuf[i, 0:16]) * third
                    abuf[i, 16:32] = (abuf[i, 16:32] + bbuf[i, 16:32]
                                      + cbuf[i, 16:32]) * third
                    return 0

                lax.fori_loop(0, 64, rowm, 0, unroll=2)
            pltpu.async_copy(abuf, dst.at[pl.ds(r0, 64)], ss0)
            pltpu.async_copy(zbuf, acc.at[pl.ds(r0, 64)], gs0)
            return 0

        def chunk_w(k, _):
            pltpu.make_async_copy(abuf, dst.at[pl.ds(r0_tile, 64)],
                                  ss0).wait()
            pltpu.make_async_copy(zbuf, acc.at[pl.ds(r0_tile, 64)],
                                  gs0).wait()
            chunk(k, 0)
            return 0

        chunk(0, 0)
        lax.fori_loop(1, 49, chunk_w, 0)
        pltpu.make_async_copy(abuf, dst.at[pl.ds(r0_tile, 64)], ss0).wait()
        pltpu.make_async_copy(zbuf, acc.at[pl.ds(r0_tile, 64)], gs0).wait()

    def _round(r, _):
        c = 2 * r + sc
        src0 = emb4.at[c]
        y = ybuf.at[c]
        e1 = e1buf.at[c]
        _spmm(src0, rows2d, cols2d)
        plsc.subcore_barrier()
        _drain(y, mode=1, l=0)
        plsc.subcore_barrier()
        _spmm(y, cols2d, rows2d)
        plsc.subcore_barrier()
        _drain(e1, mode=0)
        plsc.subcore_barrier()
        _spmm(e1, rows2d, cols2d)
        plsc.subcore_barrier()
        _drain(y, mode=1, l=1)
        plsc.subcore_barrier()
        _spmm(y, cols2d, rows2d)
        plsc.subcore_barrier()
        _drain(y, mode=2, s0=src0, s1=e1)
        plsc.subcore_barrier()
        s0i = t * _SRT
        for gp in range(2):
            for s in range(4):
                pltpu.async_copy(sel2d.at[s0i + gp * 4 + s], GI[s], ISEM[s])
            for s in range(4):
                pltpu.make_async_copy(sel2d.at[s0i], GI[s], ISEM[s]).wait()
                pltpu.async_copy(y.at[GI[s]], GB[s], GSEM[s])
            for s in range(4):
                pltpu.make_async_copy(y.at[GI[s]], GB[s], GSEM[s]).wait()
                pltpu.sync_copy(GB[s], sel_out.at[c].at[s0i + gp * 4 + s])
        plsc.subcore_barrier()
        return 0

    lax.fori_loop(0, 2, _round, 0)


def _sc_propagate(emb4, filt3, rows2d, cols2d, vals2d, sel2d):
    mesh = plsc.VectorSubcoreMesh(core_axis_name="c", subcore_axis_name="s")
    f32 = jnp.float32
    out_type = (
        jax.ShapeDtypeStruct((_NCH, _SR, 128, _DC), f32),
        jax.ShapeDtypeStruct((_NCH, _NP, _DC), f32),
        jax.ShapeDtypeStruct((_NCH, _NP, _DC), f32),
    )
    scratch = (
        [pltpu.VMEM_SHARED((_NP, _DC), f32)]
        + [pltpu.VMEM((128,), jnp.int32)] * 8
        + [pltpu.VMEM((128,), f32)] * 4
        + [pltpu.VMEM((128, _DC), f32)] * 4
        + [pltpu.VMEM((64, _DC), f32)]
        + [pltpu.VMEM((64, _DC), f32)] * 3
        + [pltpu.VMEM((64,), f32)]
        + [pltpu.SemaphoreType.DMA] * 12
    )
    run = pl.kernel(_sc_body, out_type=out_type, mesh=mesh,
                    scratch_types=scratch,
                    compiler_params=pltpu.CompilerParams(
                        use_tc_tiling_on_sc=False))
    sel4, _, _ = run(emb4, filt3, rows2d, cols2d, vals2d, sel2d)
    return sel4


def _loss_body(ue_ref, bp_ref, bn_ref, out_ref):
    i = pl.program_id(0)

    @pl.when(i == 0)
    def _():
        out_ref[0, 0] = 0.0

    x = jnp.sum(ue_ref[...] * (bn_ref[...] - bp_ref[...]), axis=1)
    sp = jnp.maximum(x, 0.0) + jnp.log1p(jnp.exp(-jnp.abs(x)))
    out_ref[0, 0] += jnp.sum(sp)

    @pl.when(i == pl.num_programs(0) - 1)
    def _():
        out_ref[0, 0] = out_ref[0, 0] * (1.0 / _BATCH)


def _l2_body(e_ref, out_ref):
    i = pl.program_id(0)

    @pl.when(i == 0)
    def _():
        out_ref[0, 0] = 0.0

    x = e_ref[...]
    out_ref[0, 0] += jnp.sum(x * x)

    @pl.when(i == pl.num_programs(0) - 1)
    def _():
        out_ref[0, 0] = out_ref[0, 0] * (0.5 / _NU)


def _tc_loss(ue, bp, bn):
    blk = 512
    grid = _BATCH // blk
    out = pl.pallas_call(
        _loss_body,
        grid=(grid,),
        in_specs=[pl.BlockSpec((blk, _D), lambda i: (i, 0))] * 3,
        out_specs=pl.BlockSpec(memory_space=pltpu.SMEM),
        out_shape=jax.ShapeDtypeStruct((1, 1), jnp.float32),
    )(ue, bp, bn)
    return out[0, 0]


def _tc_l2(all_emb):
    blk = 2000
    grid = _N // blk
    out = pl.pallas_call(
        _l2_body,
        grid=(grid,),
        in_specs=[pl.BlockSpec((blk, _D), lambda i: (i, 0))],
        out_specs=pl.BlockSpec(memory_space=pltpu.SMEM),
        out_shape=jax.ShapeDtypeStruct((1, 1), jnp.float32),
    )(all_emb)
    return out[0, 0]


def kernel(u, b, emb_u, emb_b, filt, graph_rows, graph_cols, graph_vals):
    f32 = jnp.float32
    all_emb = jnp.concatenate([emb_u, emb_b], axis=0)
    emb_pad = jnp.pad(all_emb, ((0, _NP - _N), (0, 0)))
    emb4 = emb_pad.reshape(_NP, _NCH, _DC).transpose(1, 0, 2)

    filt3 = jnp.pad(filt, ((0, 0), (0, _NP - _N))).reshape(
        2, _NP // 64, 64)

    pad_e = _ERP * 128 - graph_rows.shape[0]
    rows2d = jnp.pad(graph_rows, (0, pad_e)).reshape(_ERP, 128)
    cols2d = jnp.pad(graph_cols, (0, pad_e)).reshape(_ERP, 128)
    vals2d = jnp.pad(graph_vals, (0, pad_e)).reshape(_ERP, 128)

    sel = jnp.concatenate([u[:, 0], b[:, 0] + _NU, b[:, 1] + _NU])
    sel2d = jnp.pad(sel.astype(jnp.int32), (0, _SELP - _SEL)).reshape(
        _SR, 128)

    sel4 = _sc_propagate(emb4, filt3, rows2d, cols2d, vals2d, sel2d)
    sel_rows = sel4.reshape(_NCH, _SELP, _DC)[:, :_SEL]
    sel_rows = sel_rows.transpose(1, 0, 2).reshape(_SEL, _D)
    ue = sel_rows[:_BATCH]
    bp = sel_rows[_BATCH:2 * _BATCH]
    bn = sel_rows[2 * _BATCH:]

    loss = _tc_loss(ue, bp, bn)
    l2 = _tc_l2(all_emb)
    return (loss.astype(f32), l2.astype(f32))
